# Initial kernel scaffold; baseline (speedup 1.0000x reference)
import jax
import jax.numpy as jnp
from jax.experimental import pallas as pl


def kernel(pred, target):
    raise NotImplementedError("write your pallas kernel here")


# speedup vs baseline: 1.9626x; 1.9626x over previous
import functools

import jax
import jax.numpy as jnp
from jax import lax
from jax.experimental import pallas as pl
from jax.experimental.pallas import tpu as pltpu
from jax.experimental.pallas import tpu_sc as plsc

NC = 2
NS = 16
L = 16
NW = NC * NS
BINS = 10


def _sc_histogram(pred_flat, target, rows_w, chunk_rows):
    n_chunks = rows_w // chunk_rows
    vecs_per_chunk = chunk_rows // L
    mesh = plsc.VectorSubcoreMesh(
        core_axis_name="c", subcore_axis_name="s",
        num_cores=NC, num_subcores=NS)

    @functools.partial(
        pl.kernel,
        out_type=jax.ShapeDtypeStruct((2, NW, BINS * L), jnp.float32),
        mesh=mesh,
        scratch_types=[
            pltpu.VMEM((2 * chunk_rows,), jnp.float32),
            pltpu.VMEM((chunk_rows,), jnp.int32),
            pltpu.VMEM((BINS * L,), jnp.float32),
            pltpu.VMEM((BINS * L,), jnp.float32),
            pltpu.VMEM((BINS * L,), jnp.float32),
            pltpu.VMEM((BINS * L,), jnp.float32),
        ],
        compiler_params=pltpu.CompilerParams(needs_layout_passes=False),
    )
    def hist(pred_hbm, tgt_hbm, out_hbm, buf_p, buf_t, cnt0, cnt1, ps0, ps1):
        wid = lax.axis_index("c") * NS + lax.axis_index("s")
        lane = lax.iota(jnp.int32, L)
        lane2 = lane * 2
        zeros = jnp.zeros((L,), jnp.float32)
        ones = jnp.full((L,), 1.0, jnp.float32)
        onef = jnp.full((L,), 1.0, jnp.float32)
        tenf = jnp.full((L,), 10.0, jnp.float32)
        nine = jnp.full((L,), BINS - 1, jnp.int32)

        for b in range(BINS):
            cnt0[pl.ds(b * L, L)] = zeros
            cnt1[pl.ds(b * L, L)] = zeros
            ps0[pl.ds(b * L, L)] = zeros
            ps1[pl.ds(b * L, L)] = zeros

        row_base = wid * rows_w

        def chunk_body(c, _):
            pbase = (row_base + c * chunk_rows) * 2
            tbase = row_base + c * chunk_rows
            pltpu.sync_copy(pred_hbm.at[pl.ds(pbase, 2 * chunk_rows)], buf_p)
            pltpu.sync_copy(tgt_hbm.at[pl.ds(tbase, chunk_rows)], buf_t)

            def vec_body(j, _):
                idx0 = lane2 + j * 32
                idx1 = idx0 + 1
                p0 = plsc.load_gather(buf_p, [idx0])
                p1 = plsc.load_gather(buf_p, [idx1])
                t = plsc.load_gather(buf_t, [lane + j * L])
                m0 = t == 0
                g0 = jnp.where(m0, onef - p0, p0)
                g1 = jnp.where(m0, p1, onef - p1)
                p = jnp.where(m0, p0, p1)
                i0 = jnp.minimum((g0 * tenf).astype(jnp.int32), nine)
                i1 = jnp.minimum((g1 * tenf).astype(jnp.int32), nine)
                a0 = i0 * L + lane
                a1 = i1 * L + lane
                plsc.addupdate_scatter(cnt0, [a0], ones)
                plsc.addupdate_scatter(ps0, [a0], p)
                plsc.addupdate_scatter(cnt1, [a1], ones)
                plsc.addupdate_scatter(ps1, [a1], p)
                return 0

            lax.fori_loop(0, vecs_per_chunk, vec_body, 0)
            return 0

        lax.fori_loop(0, n_chunks, chunk_body, 0)

        for b in range(BINS):
            cnt0[pl.ds(b * L, L)] = cnt0[pl.ds(b * L, L)] + cnt1[pl.ds(b * L, L)]
            ps0[pl.ds(b * L, L)] = ps0[pl.ds(b * L, L)] + ps1[pl.ds(b * L, L)]
        pltpu.sync_copy(cnt0, out_hbm.at[0, wid])
        pltpu.sync_copy(ps0, out_hbm.at[1, wid])

    return hist(pred_flat, target)


def _combine_body(part_ref, out_ref):
    x = part_ref[...]
    cnt_t = jnp.sum(x[0], axis=(0, 2))
    ps_t = jnp.sum(x[1], axis=(0, 2))
    nz = cnt_t > 0.0
    n = jnp.sum(nz.astype(jnp.float32))
    inv = jnp.where(nz, 1.0 / jnp.where(nz, cnt_t, 1.0), 0.0)
    total = jnp.sum(ps_t * inv)
    out_ref[0, 0] = jnp.where(n > 0.0, -total / (2.0 * n), 0.0)


def kernel(pred, target):
    n_rows = pred.shape[0]
    rows_w = n_rows // NW
    chunk_rows = min(rows_w, 16384)
    pred_flat = pred.reshape(-1)
    partials = _sc_histogram(pred_flat, target, rows_w, chunk_rows)
    partials = partials.reshape(2, NW, BINS, L)
    out = pl.pallas_call(
        _combine_body,
        out_shape=jax.ShapeDtypeStruct((1, 1), jnp.float32),
        in_specs=[pl.BlockSpec(memory_space=pltpu.VMEM)],
        out_specs=pl.BlockSpec(memory_space=pltpu.SMEM),
    )(partials)
    return out[0, 0]


# speedup vs baseline: 45.2166x; 23.0390x over previous
import functools

import jax
import jax.numpy as jnp
from jax import lax
from jax.experimental import pallas as pl
from jax.experimental.pallas import tpu as pltpu
from jax.experimental.pallas import tpu_sc as plsc

NC = 2
NS = 16
L = 16
NW = NC * NS
BINS = 10


def _sc_histogram(p0_flat, p1_flat, target, rows_w, chunk_rows):
    n_chunks = rows_w // chunk_rows
    vecs_per_chunk = chunk_rows // L
    mesh = plsc.VectorSubcoreMesh(
        core_axis_name="c", subcore_axis_name="s",
        num_cores=NC, num_subcores=NS)

    @functools.partial(
        pl.kernel,
        out_type=jax.ShapeDtypeStruct((2, NW, BINS * L), jnp.float32),
        mesh=mesh,
        scratch_types=[
            pltpu.VMEM((chunk_rows,), jnp.float32),
            pltpu.VMEM((chunk_rows,), jnp.float32),
            pltpu.VMEM((chunk_rows,), jnp.int32),
            pltpu.VMEM((BINS * L,), jnp.float32),
            pltpu.VMEM((BINS * L,), jnp.float32),
            pltpu.VMEM((BINS * L,), jnp.float32),
            pltpu.VMEM((BINS * L,), jnp.float32),
        ],
        compiler_params=pltpu.CompilerParams(needs_layout_passes=False),
    )
    def hist(p0_hbm, p1_hbm, tgt_hbm, out_hbm,
             buf_p0, buf_p1, buf_t, cnt0, cnt1, ps0, ps1):
        wid = lax.axis_index("c") * NS + lax.axis_index("s")
        lane = lax.iota(jnp.int32, L)
        zeros = jnp.zeros((L,), jnp.float32)
        ones = jnp.full((L,), 1.0, jnp.float32)
        onef = jnp.full((L,), 1.0, jnp.float32)
        tenf = jnp.full((L,), 10.0, jnp.float32)
        nine = jnp.full((L,), BINS - 1, jnp.int32)

        for b in range(BINS):
            cnt0[pl.ds(b * L, L)] = zeros
            cnt1[pl.ds(b * L, L)] = zeros
            ps0[pl.ds(b * L, L)] = zeros
            ps1[pl.ds(b * L, L)] = zeros

        row_base = wid * rows_w

        def chunk_body(c, _):
            base = row_base + c * chunk_rows
            pltpu.sync_copy(p0_hbm.at[pl.ds(base, chunk_rows)], buf_p0)
            pltpu.sync_copy(p1_hbm.at[pl.ds(base, chunk_rows)], buf_p1)
            pltpu.sync_copy(tgt_hbm.at[pl.ds(base, chunk_rows)], buf_t)

            def vec_body(j, _):
                sl = pl.ds(j * L, L)
                p0 = buf_p0[sl]
                p1 = buf_p1[sl]
                t = buf_t[sl]
                m0 = t == 0
                g0 = jnp.where(m0, onef - p0, p0)
                g1 = jnp.where(m0, p1, onef - p1)
                p = jnp.where(m0, p0, p1)
                i0 = jnp.minimum((g0 * tenf).astype(jnp.int32), nine)
                i1 = jnp.minimum((g1 * tenf).astype(jnp.int32), nine)
                a0 = i0 * L + lane
                a1 = i1 * L + lane
                plsc.addupdate_scatter(cnt0, [a0], ones)
                plsc.addupdate_scatter(ps0, [a0], p)
                plsc.addupdate_scatter(cnt1, [a1], ones)
                plsc.addupdate_scatter(ps1, [a1], p)
                return 0

            lax.fori_loop(0, vecs_per_chunk, vec_body, 0)
            return 0

        lax.fori_loop(0, n_chunks, chunk_body, 0)

        for b in range(BINS):
            cnt0[pl.ds(b * L, L)] = cnt0[pl.ds(b * L, L)] + cnt1[pl.ds(b * L, L)]
            ps0[pl.ds(b * L, L)] = ps0[pl.ds(b * L, L)] + ps1[pl.ds(b * L, L)]
        pltpu.sync_copy(cnt0, out_hbm.at[0, wid])
        pltpu.sync_copy(ps0, out_hbm.at[1, wid])

    return hist(p0_flat, p1_flat, target)


def _combine_body(part_ref, out_ref):
    x = part_ref[...]
    cnt_t = jnp.sum(x[0], axis=(0, 2))
    ps_t = jnp.sum(x[1], axis=(0, 2))
    nz = cnt_t > 0.0
    n = jnp.sum(nz.astype(jnp.float32))
    inv = jnp.where(nz, 1.0 / jnp.where(nz, cnt_t, 1.0), 0.0)
    total = jnp.sum(ps_t * inv)
    out_ref[0, 0] = jnp.where(n > 0.0, -total / (2.0 * n), 0.0)


def kernel(pred, target):
    n_rows = pred.shape[0]
    rows_w = n_rows // NW
    chunk_rows = min(rows_w, 16384)
    p0 = lax.slice(pred, (0, 0), (n_rows, 1)).reshape(n_rows)
    p1 = lax.slice(pred, (0, 1), (n_rows, 2)).reshape(n_rows)
    partials = _sc_histogram(p0, p1, target, rows_w, chunk_rows)
    partials = partials.reshape(2, NW, BINS, L)
    out = pl.pallas_call(
        _combine_body,
        out_shape=jax.ShapeDtypeStruct((1, 1), jnp.float32),
        in_specs=[pl.BlockSpec(memory_space=pltpu.VMEM)],
        out_specs=pl.BlockSpec(memory_space=pltpu.SMEM),
    )(partials)
    return out[0, 0]


# speedup vs baseline: 51.5007x; 1.1390x over previous
import functools

import jax
import jax.numpy as jnp
from jax import lax
from jax.experimental import pallas as pl
from jax.experimental.pallas import tpu as pltpu
from jax.experimental.pallas import tpu_sc as plsc

NC = 2
NS = 16
L = 16
NW = NC * NS
BINS = 10


def _sc_histogram(p0_flat, p1_flat, target, rows_w, chunk_rows):
    n_chunks = rows_w // chunk_rows
    unroll = 4
    iters = chunk_rows // L // unroll
    mesh = plsc.VectorSubcoreMesh(
        core_axis_name="c", subcore_axis_name="s",
        num_cores=NC, num_subcores=NS)

    @functools.partial(
        pl.kernel,
        out_type=jax.ShapeDtypeStruct((2, NW, BINS * L), jnp.float32),
        mesh=mesh,
        scratch_types=[
            pltpu.VMEM((chunk_rows,), jnp.float32),
            pltpu.VMEM((chunk_rows,), jnp.float32),
            pltpu.VMEM((chunk_rows,), jnp.int32),
            pltpu.VMEM((chunk_rows,), jnp.float32),
            pltpu.VMEM((chunk_rows,), jnp.float32),
            pltpu.VMEM((chunk_rows,), jnp.int32),
            pltpu.VMEM((BINS * L,), jnp.float32),
            pltpu.VMEM((BINS * L,), jnp.float32),
            pltpu.VMEM((BINS * L,), jnp.float32),
            pltpu.VMEM((BINS * L,), jnp.float32),
            pltpu.SemaphoreType.DMA,
            pltpu.SemaphoreType.DMA,
        ],
        compiler_params=pltpu.CompilerParams(needs_layout_passes=False),
    )
    def hist(p0_hbm, p1_hbm, tgt_hbm, out_hbm,
             p0a, p1a, ta, p0b, p1b, tb, cnt0, cnt1, ps0, ps1, sema, semb):
        wid = lax.axis_index("c") * NS + lax.axis_index("s")
        lane = lax.iota(jnp.int32, L)
        zeros = jnp.zeros((L,), jnp.float32)
        ones = jnp.full((L,), 1.0, jnp.float32)
        onef = jnp.full((L,), 1.0, jnp.float32)
        tenf = jnp.full((L,), 10.0, jnp.float32)
        ninef = jnp.full((L,), float(BINS - 1), jnp.float32)

        for b in range(BINS):
            cnt0[pl.ds(b * L, L)] = zeros
            cnt1[pl.ds(b * L, L)] = zeros
            ps0[pl.ds(b * L, L)] = zeros
            ps1[pl.ds(b * L, L)] = zeros

        row_base = wid * rows_w
        bufs = [(p0a, p1a, ta, sema), (p0b, p1b, tb, semb)]

        def start(c, buf):
            bp0, bp1, bt, sem = buf
            base = row_base + c * chunk_rows
            return (
                pltpu.async_copy(p0_hbm.at[pl.ds(base, chunk_rows)], bp0, sem),
                pltpu.async_copy(p1_hbm.at[pl.ds(base, chunk_rows)], bp1, sem),
                pltpu.async_copy(tgt_hbm.at[pl.ds(base, chunk_rows)], bt, sem),
            )

        pend = start(0, bufs[0])
        for c in range(n_chunks):
            bp0, bp1, bt, _ = bufs[c % 2]
            nxt = start(c + 1, bufs[(c + 1) % 2]) if c + 1 < n_chunks else None
            for d in pend:
                d.wait()

            def vec_body(j, _, bp0=bp0, bp1=bp1, bt=bt):
                for u in range(unroll):
                    sl = pl.ds(j * (unroll * L) + u * L, L)
                    p0 = bp0[sl]
                    p1 = bp1[sl]
                    t = bt[sl]
                    m0 = t == 0
                    g0 = jnp.where(m0, onef - p0, p0)
                    g1 = jnp.where(m0, p1, onef - p1)
                    p = jnp.where(m0, p0, p1)
                    i0 = jnp.minimum(g0 * tenf, ninef).astype(jnp.int32)
                    i1 = jnp.minimum(g1 * tenf, ninef).astype(jnp.int32)
                    a0 = i0 * L + lane
                    a1 = i1 * L + lane
                    plsc.addupdate_scatter(cnt0, [a0], ones)
                    plsc.addupdate_scatter(ps0, [a0], p)
                    plsc.addupdate_scatter(cnt1, [a1], ones)
                    plsc.addupdate_scatter(ps1, [a1], p)
                return 0

            lax.fori_loop(0, iters, vec_body, 0)
            pend = nxt

        for b in range(BINS):
            cnt0[pl.ds(b * L, L)] = cnt0[pl.ds(b * L, L)] + cnt1[pl.ds(b * L, L)]
            ps0[pl.ds(b * L, L)] = ps0[pl.ds(b * L, L)] + ps1[pl.ds(b * L, L)]
        pltpu.sync_copy(cnt0, out_hbm.at[0, wid])
        pltpu.sync_copy(ps0, out_hbm.at[1, wid])

    return hist(p0_flat, p1_flat, target)


def _combine_body(part_ref, out_ref):
    x = part_ref[...]
    cnt_t = jnp.sum(x[0], axis=(0, 2))
    ps_t = jnp.sum(x[1], axis=(0, 2))
    nz = cnt_t > 0.0
    n = jnp.sum(nz.astype(jnp.float32))
    inv = jnp.where(nz, 1.0 / jnp.where(nz, cnt_t, 1.0), 0.0)
    total = jnp.sum(ps_t * inv)
    out_ref[0, 0] = jnp.where(n > 0.0, -total / (2.0 * n), 0.0)


def kernel(pred, target):
    n_rows = pred.shape[0]
    rows_w = n_rows // NW
    chunk_rows = min(rows_w, 8192)
    p0 = lax.slice(pred, (0, 0), (n_rows, 1)).reshape(n_rows)
    p1 = lax.slice(pred, (0, 1), (n_rows, 2)).reshape(n_rows)
    partials = _sc_histogram(p0, p1, target, rows_w, chunk_rows)
    partials = partials.reshape(2, NW, BINS, L)
    out = pl.pallas_call(
        _combine_body,
        out_shape=jax.ShapeDtypeStruct((1, 1), jnp.float32),
        in_specs=[pl.BlockSpec(memory_space=pltpu.VMEM)],
        out_specs=pl.BlockSpec(memory_space=pltpu.SMEM),
    )(partials)
    return out[0, 0]


# speedup vs baseline: 88.6589x; 1.7215x over previous
import functools

import jax
import jax.numpy as jnp
from jax import lax
from jax.experimental import pallas as pl
from jax.experimental.pallas import tpu as pltpu
from jax.experimental.pallas import tpu_sc as plsc

NC = 2
NS = 16
L = 16
NW = NC * NS
BINS = 10


def _sc_histogram(p0_flat, p1_flat, target, rows_w, chunk_rows):
    n_chunks = rows_w // chunk_rows
    unroll = 8
    mesh = plsc.VectorSubcoreMesh(
        core_axis_name="c", subcore_axis_name="s",
        num_cores=NC, num_subcores=NS)

    @functools.partial(
        pl.kernel,
        out_type=jax.ShapeDtypeStruct((2, NW, BINS * L), jnp.float32),
        mesh=mesh,
        scratch_types=[
            pltpu.VMEM((chunk_rows,), jnp.float32),
            pltpu.VMEM((chunk_rows,), jnp.float32),
            pltpu.VMEM((chunk_rows,), jnp.int32),
            pltpu.VMEM((chunk_rows,), jnp.float32),
            pltpu.VMEM((chunk_rows,), jnp.float32),
            pltpu.VMEM((chunk_rows,), jnp.int32),
            pltpu.VMEM((BINS * L,), jnp.float32),
            pltpu.VMEM((BINS * L,), jnp.float32),
            pltpu.VMEM((BINS * L,), jnp.float32),
            pltpu.VMEM((BINS * L,), jnp.float32),
            pltpu.SemaphoreType.DMA,
            pltpu.SemaphoreType.DMA,
        ],
        compiler_params=pltpu.CompilerParams(needs_layout_passes=False),
    )
    def hist(p0_hbm, p1_hbm, tgt_hbm, out_hbm,
             p0a, p1a, ta, p0b, p1b, tb, cnt0, cnt1, ps0, ps1, sema, semb):
        wid = lax.axis_index("c") * NS + lax.axis_index("s")
        lane = lax.iota(jnp.int32, L)
        zeros = jnp.zeros((L,), jnp.float32)
        ones = jnp.full((L,), 1.0, jnp.float32)
        onef = jnp.full((L,), 1.0, jnp.float32)
        tenf = jnp.full((L,), 10.0, jnp.float32)
        ninef = jnp.full((L,), float(BINS - 1), jnp.float32)

        for b in range(BINS):
            cnt0[pl.ds(b * L, L)] = zeros
            cnt1[pl.ds(b * L, L)] = zeros
            ps0[pl.ds(b * L, L)] = zeros
            ps1[pl.ds(b * L, L)] = zeros

        row_base = wid * rows_w
        bufs = [(p0a, p1a, ta, sema), (p0b, p1b, tb, semb)]

        def start(c, buf):
            bp0, bp1, bt, sem = buf
            base = row_base + c * chunk_rows
            return (
                pltpu.async_copy(p0_hbm.at[pl.ds(base, chunk_rows)], bp0, sem),
                pltpu.async_copy(p1_hbm.at[pl.ds(base, chunk_rows)], bp1, sem),
                pltpu.async_copy(tgt_hbm.at[pl.ds(base, chunk_rows)], bt, sem),
            )

        pend = start(0, bufs[0])
        for c in range(n_chunks):
            bp0, bp1, bt, _ = bufs[c % 2]
            nxt = start(c + 1, bufs[(c + 1) % 2]) if c + 1 < n_chunks else None
            for d in pend:
                d.wait()

            @plsc.parallel_loop(0, chunk_rows // L, 1, unroll=unroll)
            def _vec_body(j, bp0=bp0, bp1=bp1, bt=bt):
                sl = pl.ds(j * L, L)
                p0 = bp0[sl]
                p1 = bp1[sl]
                t = bt[sl]
                m0 = t == 0
                g0 = jnp.where(m0, onef - p0, p0)
                g1 = jnp.where(m0, p1, onef - p1)
                p = jnp.where(m0, p0, p1)
                i0 = jnp.minimum(g0 * tenf, ninef).astype(jnp.int32)
                i1 = jnp.minimum(g1 * tenf, ninef).astype(jnp.int32)
                a0 = i0 * L + lane
                a1 = i1 * L + lane
                plsc.addupdate_scatter(cnt0, [a0], ones)
                plsc.addupdate_scatter(ps0, [a0], p)
                plsc.addupdate_scatter(cnt1, [a1], ones)
                plsc.addupdate_scatter(ps1, [a1], p)

            pend = nxt

        for b in range(BINS):
            cnt0[pl.ds(b * L, L)] = cnt0[pl.ds(b * L, L)] + cnt1[pl.ds(b * L, L)]
            ps0[pl.ds(b * L, L)] = ps0[pl.ds(b * L, L)] + ps1[pl.ds(b * L, L)]
        pltpu.sync_copy(cnt0, out_hbm.at[0, wid])
        pltpu.sync_copy(ps0, out_hbm.at[1, wid])

    return hist(p0_flat, p1_flat, target)


def _combine_body(part_ref, out_ref):
    x = part_ref[...]
    cnt_t = jnp.sum(x[0], axis=(0, 2))
    ps_t = jnp.sum(x[1], axis=(0, 2))
    nz = cnt_t > 0.0
    n = jnp.sum(nz.astype(jnp.float32))
    inv = jnp.where(nz, 1.0 / jnp.where(nz, cnt_t, 1.0), 0.0)
    total = jnp.sum(ps_t * inv)
    out_ref[0, 0] = jnp.where(n > 0.0, -total / (2.0 * n), 0.0)


def kernel(pred, target):
    n_rows = pred.shape[0]
    rows_w = n_rows // NW
    chunk_rows = min(rows_w, 8192)
    p0 = lax.slice(pred, (0, 0), (n_rows, 1)).reshape(n_rows)
    p1 = lax.slice(pred, (0, 1), (n_rows, 2)).reshape(n_rows)
    partials = _sc_histogram(p0, p1, target, rows_w, chunk_rows)
    partials = partials.reshape(2, NW, BINS, L)
    out = pl.pallas_call(
        _combine_body,
        out_shape=jax.ShapeDtypeStruct((1, 1), jnp.float32),
        in_specs=[pl.BlockSpec(memory_space=pltpu.VMEM)],
        out_specs=pl.BlockSpec(memory_space=pltpu.SMEM),
    )(partials)
    return out[0, 0]


# speedup vs baseline: 96.0818x; 1.0837x over previous
import functools

import jax
import jax.numpy as jnp
from jax import lax
from jax.experimental import pallas as pl
from jax.experimental.pallas import tpu as pltpu
from jax.experimental.pallas import tpu_sc as plsc

NC = 2
NS = 16
L = 16
NW = NC * NS
BINS = 10


def _sc_histogram(pred_blocks, target, rows_w, chunk_rows):
    n_chunks = rows_w // chunk_rows
    unroll = 8
    mesh = plsc.VectorSubcoreMesh(
        core_axis_name="c", subcore_axis_name="s",
        num_cores=NC, num_subcores=NS)

    @functools.partial(
        pl.kernel,
        out_type=jax.ShapeDtypeStruct((2, NW, BINS * L), jnp.float32),
        mesh=mesh,
        scratch_types=[
            pltpu.VMEM((2 * chunk_rows,), jnp.float32),
            pltpu.VMEM((chunk_rows,), jnp.int32),
            pltpu.VMEM((2 * chunk_rows,), jnp.float32),
            pltpu.VMEM((chunk_rows,), jnp.int32),
            pltpu.VMEM((BINS * L,), jnp.float32),
            pltpu.VMEM((BINS * L,), jnp.float32),
            pltpu.VMEM((BINS * L,), jnp.float32),
            pltpu.VMEM((BINS * L,), jnp.float32),
            pltpu.SemaphoreType.DMA,
            pltpu.SemaphoreType.DMA,
        ],
        compiler_params=pltpu.CompilerParams(needs_layout_passes=False),
    )
    def hist(pred_hbm, tgt_hbm, out_hbm,
             pa, ta, pb, tb, cnt0, cnt1, ps0, ps1, sema, semb):
        wid = lax.axis_index("c") * NS + lax.axis_index("s")
        lane = lax.iota(jnp.int32, L)
        lane10 = lane * BINS
        zeros = jnp.zeros((L,), jnp.float32)
        ones = jnp.full((L,), 1.0, jnp.float32)
        onef = jnp.full((L,), 1.0, jnp.float32)
        tenf = jnp.full((L,), 10.0, jnp.float32)
        ninef = jnp.full((L,), float(BINS - 1), jnp.float32)

        for b in range(BINS):
            cnt0[pl.ds(b * L, L)] = zeros
            cnt1[pl.ds(b * L, L)] = zeros
            ps0[pl.ds(b * L, L)] = zeros
            ps1[pl.ds(b * L, L)] = zeros

        row_base = wid * rows_w
        bufs = [(pa, ta, sema), (pb, tb, semb)]

        def start(c, buf):
            bp, bt, sem = buf
            base = row_base + c * chunk_rows
            return (
                pltpu.async_copy(
                    pred_hbm.at[pl.ds(2 * base, 2 * chunk_rows)], bp, sem),
                pltpu.async_copy(tgt_hbm.at[pl.ds(base, chunk_rows)], bt, sem),
            )

        pend = start(0, bufs[0])
        for c in range(n_chunks):
            bp, bt, _ = bufs[c % 2]
            nxt = start(c + 1, bufs[(c + 1) % 2]) if c + 1 < n_chunks else None
            for d in pend:
                d.wait()

            @plsc.parallel_loop(0, chunk_rows // L, 1, unroll=unroll)
            def _vec_body(v, bp=bp, bt=bt):
                off0 = (v >> 3) * (2 * 128) + (v & 7) * L
                p0 = bp[pl.ds(off0, L)]
                p1 = bp[pl.ds(off0 + 128, L)]
                t = bt[pl.ds(v * L, L)]
                m0 = t == 0
                g0 = jnp.where(m0, onef - p0, p0)
                g1 = jnp.where(m0, p1, onef - p1)
                p = jnp.where(m0, p0, p1)
                i0 = jnp.minimum(g0 * tenf, ninef).astype(jnp.int32)
                i1 = jnp.minimum(g1 * tenf, ninef).astype(jnp.int32)
                a0 = i0 + lane10
                a1 = i1 + lane10
                plsc.addupdate_scatter(cnt0, [a0], ones)
                plsc.addupdate_scatter(ps0, [a0], p)
                plsc.addupdate_scatter(cnt1, [a1], ones)
                plsc.addupdate_scatter(ps1, [a1], p)

            pend = nxt

        for b in range(BINS):
            cnt0[pl.ds(b * L, L)] = cnt0[pl.ds(b * L, L)] + cnt1[pl.ds(b * L, L)]
            ps0[pl.ds(b * L, L)] = ps0[pl.ds(b * L, L)] + ps1[pl.ds(b * L, L)]
        pltpu.sync_copy(cnt0, out_hbm.at[0, wid])
        pltpu.sync_copy(ps0, out_hbm.at[1, wid])

    return hist(pred_blocks, target)


def _combine_body(part_ref, out_ref):
    x = part_ref[...]
    cnt_t = jnp.sum(x[0], axis=(0, 1))
    ps_t = jnp.sum(x[1], axis=(0, 1))
    nz = cnt_t > 0.0
    n = jnp.sum(nz.astype(jnp.float32))
    inv = jnp.where(nz, 1.0 / jnp.where(nz, cnt_t, 1.0), 0.0)
    total = jnp.sum(ps_t * inv)
    out_ref[0, 0] = jnp.where(n > 0.0, -total / (2.0 * n), 0.0)


def kernel(pred, target):
    n_rows = pred.shape[0]
    rows_w = n_rows // NW
    chunk_rows = min(rows_w, 8192)
    pred_blocks = pred.reshape(n_rows // 128, 128, 2).transpose(0, 2, 1)
    pred_blocks = pred_blocks.reshape(2 * n_rows)
    partials = _sc_histogram(pred_blocks, target, rows_w, chunk_rows)
    partials = partials.reshape(2, NW, L, BINS)
    out = pl.pallas_call(
        _combine_body,
        out_shape=jax.ShapeDtypeStruct((1, 1), jnp.float32),
        in_specs=[pl.BlockSpec(memory_space=pltpu.VMEM)],
        out_specs=pl.BlockSpec(memory_space=pltpu.SMEM),
    )(partials)
    return out[0, 0]
